# baseline (speedup 1.0000x reference)
import dataclasses
import functools

import jax
import jax.numpy as jnp
from jax import lax
from jax.experimental import pallas as pl
from jax.experimental.pallas import tpu as pltpu
from jax.experimental.pallas import tpu_sc as plsc

NC = 2
NS = 16
NW = NC * NS
L = 16
D = 64
PAIR = 2 * D
GAMMA = 12.0

CHUNK = 128


def _sc_score(src2, spar, rel2, rpar, dst2, dpar, ent2, relt2, batch):
    per_w = batch // NW
    nchunk = per_w // CHUNK
    mesh = plsc.VectorSubcoreMesh(core_axis_name="c", subcore_axis_name="s")
    cp = pltpu.CompilerParams()
    if "needs_layout_passes" in pltpu.CompilerParams.__dataclass_fields__:
        cp = dataclasses.replace(cp, needs_layout_passes=False)

    @functools.partial(
        pl.kernel,
        out_type=jax.ShapeDtypeStruct((batch,), jnp.float32),
        mesh=mesh,
        compiler_params=cp,
        scratch_types=[
            pltpu.VMEM((CHUNK,), jnp.int32),
            pltpu.VMEM((CHUNK,), jnp.int32),
            pltpu.VMEM((CHUNK,), jnp.int32),
            pltpu.VMEM((CHUNK,), jnp.int32),
            pltpu.VMEM((CHUNK,), jnp.int32),
            pltpu.VMEM((CHUNK,), jnp.int32),
            pltpu.VMEM((CHUNK, PAIR), jnp.float32),
            pltpu.VMEM((CHUNK, PAIR), jnp.float32),
            pltpu.VMEM((CHUNK, PAIR), jnp.float32),
            pltpu.VMEM((CHUNK,), jnp.float32),
            pltpu.SemaphoreType.DMA,
            pltpu.SemaphoreType.DMA,
            pltpu.SemaphoreType.DMA,
        ],
    )
    def sc_kernel(src2_hbm, spar_hbm, rel2_hbm, rpar_hbm, dst2_hbm, dpar_hbm,
                  ent_hbm, relt_hbm, out_hbm,
                  si_v, ri_v, di_v, sp_s, rp_s, dp_s, h_v, r_v, t_v, s_v,
                  sem_h, sem_t, sem_r):
        wid = lax.axis_index("s") * NC + lax.axis_index("c")
        base = wid * per_w

        @pl.loop(0, nchunk)
        def _chunk(k):
            off = base + k * CHUNK
            pltpu.sync_copy(src2_hbm.at[pl.ds(off, CHUNK)], si_v)
            pltpu.sync_copy(dst2_hbm.at[pl.ds(off, CHUNK)], di_v)
            pltpu.sync_copy(rel2_hbm.at[pl.ds(off, CHUNK)], ri_v)
            pltpu.sync_copy(spar_hbm.at[pl.ds(off, CHUNK)], sp_s)
            pltpu.sync_copy(dpar_hbm.at[pl.ds(off, CHUNK)], dp_s)
            pltpu.sync_copy(rpar_hbm.at[pl.ds(off, CHUNK)], rp_s)
            cp_h = pltpu.async_copy(ent_hbm.at[si_v], h_v, sem_h)
            cp_t = pltpu.async_copy(ent_hbm.at[di_v], t_v, sem_t)
            cp_r = pltpu.async_copy(relt_hbm.at[ri_v], r_v, sem_r)
            cp_h.wait()
            cp_t.wait()
            cp_r.wait()

            lane = lax.iota(jnp.int32, L)

            @pl.loop(0, CHUNK // L)
            def _group(g):
                vec = jnp.zeros((L,), jnp.float32)
                spv = sp_s[pl.ds(g * L, L)]
                dpv = dp_s[pl.ds(g * L, L)]
                rpv = rp_s[pl.ds(g * L, L)]
                for j in range(L):
                    row = g * L + j
                    ho = spv[j]
                    to = dpv[j]
                    ro = rpv[j]
                    acc = jnp.zeros((L,), jnp.float32)
                    for c in range(D // L):
                        hv = h_v[row, pl.ds(ho + c * L, L)]
                        tv = t_v[row, pl.ds(to + c * L, L)]
                        rv = r_v[row, pl.ds(ro + c * L, L)]
                        acc = acc + jnp.abs(hv + rv - tv)
                    vec = jnp.where(lane == j, GAMMA - jnp.sum(acc), vec)
                s_v[pl.ds(g * L, L)] = vec

            pltpu.sync_copy(s_v, out_hbm.at[pl.ds(off, CHUNK)])

    return sc_kernel(src2, spar, rel2, rpar, dst2, dpar, ent2, relt2)


def kernel(src, rel, dst, mode, ent_embed, rel_embed):
    del mode
    batch = src.shape[0]
    ent2 = ent_embed.reshape(-1, PAIR)
    relt2 = rel_embed.reshape(-1, PAIR)
    src2 = lax.shift_right_logical(src, 1)
    dst2 = lax.shift_right_logical(dst, 1)
    rel2 = lax.shift_right_logical(rel, 1)
    spar = (src & 1) * D
    dpar = (dst & 1) * D
    rpar = (rel & 1) * D
    return _sc_score(src2, spar, rel2, rpar, dst2, dpar, ent2, relt2, batch)


# speedup vs baseline: 1.7876x; 1.7876x over previous
import dataclasses
import functools

import jax
import jax.numpy as jnp
from jax import lax
from jax.experimental import pallas as pl
from jax.experimental.pallas import tpu as pltpu
from jax.experimental.pallas import tpu_sc as plsc

NC = 2
NS = 16
NW = NC * NS
L = 16
D = 64
GAMMA = 12.0

CHUNK = 32


def _sc_score(sti, sro, ri2, rpo, dti, dro, ent3, rel2, batch):
    per_w = batch // NW
    nchunk = per_w // CHUNK
    mesh = plsc.VectorSubcoreMesh(core_axis_name="c", subcore_axis_name="s")
    cp = pltpu.CompilerParams()
    if "needs_layout_passes" in pltpu.CompilerParams.__dataclass_fields__:
        cp = dataclasses.replace(cp, needs_layout_passes=False)

    @functools.partial(
        pl.kernel,
        out_type=jax.ShapeDtypeStruct((batch,), jnp.float32),
        mesh=mesh,
        compiler_params=cp,
        scratch_types=[
            pltpu.VMEM((CHUNK,), jnp.int32),
            pltpu.VMEM((CHUNK,), jnp.int32),
            pltpu.VMEM((CHUNK,), jnp.int32),
            pltpu.VMEM((CHUNK,), jnp.int32),
            pltpu.VMEM((CHUNK,), jnp.int32),
            pltpu.VMEM((CHUNK,), jnp.int32),
            pltpu.VMEM((CHUNK, 8, D), jnp.float32),
            pltpu.VMEM((CHUNK, 8, D), jnp.float32),
            pltpu.VMEM((CHUNK, 2 * D), jnp.float32),
            pltpu.VMEM((CHUNK,), jnp.float32),
            pltpu.SemaphoreType.DMA,
            pltpu.SemaphoreType.DMA,
        ],
    )
    def sc_kernel(sti_hbm, sro_hbm, ri2_hbm, rpo_hbm, dti_hbm, dro_hbm,
                  ent_hbm, relt_hbm, out_hbm,
                  si_v, so_v, ri_v, rp_v, di_v, do_v, h_v, t_v, r_v, s_v,
                  sem_e, sem_r):
        wid = lax.axis_index("s") * NC + lax.axis_index("c")
        base = wid * per_w

        @pl.loop(0, nchunk)
        def _chunk(k):
            off = base + k * CHUNK
            pltpu.sync_copy(sti_hbm.at[pl.ds(off, CHUNK)], si_v)
            pltpu.sync_copy(dti_hbm.at[pl.ds(off, CHUNK)], di_v)
            pltpu.sync_copy(ri2_hbm.at[pl.ds(off, CHUNK)], ri_v)
            pltpu.sync_copy(sro_hbm.at[pl.ds(off, CHUNK)], so_v)
            pltpu.sync_copy(dro_hbm.at[pl.ds(off, CHUNK)], do_v)
            pltpu.sync_copy(rpo_hbm.at[pl.ds(off, CHUNK)], rp_v)

            cp_r = pltpu.async_copy(relt_hbm.at[ri_v], r_v, sem_r)

            pend = []
            for g in range(CHUNK // L):
                siv = si_v[pl.ds(g * L, L)]
                div = di_v[pl.ds(g * L, L)]
                for j in range(L):
                    row = g * L + j
                    pend.append(pltpu.async_copy(
                        ent_hbm.at[siv[j]], h_v.at[row], sem_e))
                    pend.append(pltpu.async_copy(
                        ent_hbm.at[div[j]], t_v.at[row], sem_e))
            for cp_ in pend:
                cp_.wait()
            cp_r.wait()

            lane = lax.iota(jnp.int32, L)

            @pl.loop(0, CHUNK // L)
            def _group(g):
                c_vec = g * L + lane
                r_s = so_v[pl.ds(g * L, L)]
                r_d = do_v[pl.ds(g * L, L)]
                p_r = rp_v[pl.ds(g * L, L)]
                acc = jnp.zeros((L,), jnp.float32)
                col = jnp.zeros((L,), jnp.int32)
                for j in range(D):
                    hv = plsc.load_gather(h_v, [c_vec, r_s, col])
                    tv = plsc.load_gather(t_v, [c_vec, r_d, col])
                    rv = plsc.load_gather(r_v, [c_vec, p_r + col])
                    acc = acc + jnp.abs(hv + rv - tv)
                    col = col + 1
                s_v[pl.ds(g * L, L)] = GAMMA - acc

            pltpu.sync_copy(s_v, out_hbm.at[pl.ds(off, CHUNK)])

    return sc_kernel(sti, sro, ri2, rpo, dti, dro, ent3, rel2)


def kernel(src, rel, dst, mode, ent_embed, rel_embed):
    del mode
    batch = src.shape[0]
    ent3 = ent_embed.reshape(-1, 8, D)
    rel2 = rel_embed.reshape(-1, 2 * D)
    sti = lax.shift_right_logical(src, 3)
    dti = lax.shift_right_logical(dst, 3)
    ri2 = lax.shift_right_logical(rel, 1)
    sro = src & 7
    dro = dst & 7
    rpo = (rel & 1) * D
    return _sc_score(sti, sro, ri2, rpo, dti, dro, ent3, rel2, batch)
